# baseline (speedup 1.0000x reference)
import functools

import jax
import jax.numpy as jnp
from jax import lax
from jax.experimental import pallas as pl
from jax.experimental.pallas import tpu as pltpu
from jax.experimental.pallas import tpu_sc as plsc

_K = 32
_SEQ = 2048
_GROUPS = _SEQ // _K


def _ce(v, i, l, desc):
    a, b = v[i], v[l]
    mx = jnp.maximum(a, b)
    mn = jnp.minimum(a, b)
    if desc:
        v[i], v[l] = mx, mn
    else:
        v[i], v[l] = mn, mx


def _oem_merge(v, lo, n, r, desc):
    m = r * 2
    if m < n:
        _oem_merge(v, lo, n, m, desc)
        _oem_merge(v, lo + r, n, m, desc)
        for i in range(lo + r, lo + n - r, m):
            _ce(v, i, i + r, desc)
    else:
        _ce(v, lo, lo + r, desc)


def _sort32_list(v, desc, lo=0, n=_K):
    if n > 1:
        m = n // 2
        _sort32_list(v, desc, lo, m)
        _sort32_list(v, desc, lo + m, m)
        _oem_merge(v, lo, n, 1, desc)


def _cleanup_list(v, desc):
    j = _K // 2
    while j >= 1:
        for i in range(_K):
            l = i ^ j
            if l > i:
                a, b = v[i], v[l]
                mx = jnp.maximum(a, b)
                mn = jnp.minimum(a, b)
                if desc:
                    v[i], v[l] = mx, mn
                else:
                    v[i], v[l] = mn, mx
        j //= 2


def _cleanup_list_masked(v, dmask):
    j = _K // 2
    while j >= 1:
        for i in range(_K):
            l = i ^ j
            if l > i:
                a, b = v[i], v[l]
                mx = jnp.maximum(a, b)
                mn = jnp.minimum(a, b)
                v[i] = jnp.where(dmask, mx, mn)
                v[l] = jnp.where(dmask, mn, mx)
        j //= 2


def _merge_lists(va, vb, desc):
    t = [jnp.maximum(va[p], vb[p]) for p in range(_K)]
    _cleanup_list(t, desc)
    return t


def _ce_split(x, j):
    g = _K // (2 * j)
    xr = x.reshape((g, 2, j) + x.shape[1:])
    return xr[:, 0], xr[:, 1], g


def _bitonic_merge(x, desc):
    j = _K // 2
    while j >= 1:
        a, b, g = _ce_split(x, j)
        mn = jnp.minimum(a, b)
        mx = jnp.maximum(a, b)
        parts = []
        for gi in range(g):
            if desc:
                parts.append(mx[gi])
                parts.append(mn[gi])
            else:
                parts.append(mn[gi])
                parts.append(mx[gi])
        x = jnp.concatenate(parts, axis=0)
        j //= 2
    return x


def _topk_pool(x):
    hb = x.shape[-1]
    x = x.reshape(_K, _GROUPS, hb)
    dmask = jax.lax.broadcasted_iota(jnp.int32, (8, 128), 0) < 4
    pmask = jax.lax.broadcasted_iota(jnp.int32, (16, 128), 0) < 8
    outs = []
    for c in range(hb // 128):
        def ptile(p, gta, gtb):
            t = x[p, gta * 8:(gta + 1) * 8, c * 128:(c + 1) * 128]
            u = x[p, gtb * 8:(gtb + 1) * 8, c * 128:(c + 1) * 128]
            return jnp.concatenate([t, u], axis=0).astype(jnp.bfloat16)

        def sorted_pack(gta, gtb, desc):
            v = [ptile(p, gta, gtb) for p in range(_K)]
            _sort32_list(v, desc)
            return v

        q01 = _merge_lists(sorted_pack(0, 1, True), sorted_pack(4, 5, False),
                           True)
        q23 = _merge_lists(sorted_pack(2, 3, True), sorted_pack(6, 7, False),
                           False)
        r = [jnp.maximum(q01[p], q23[p]) for p in range(_K)]
        _cleanup_list_masked(r, pmask)
        r = [jnp.maximum(r[p][:8], r[p][8:]).astype(jnp.float32)
             for p in range(_K)]
        _cleanup_list_masked(r, dmask)
        xs = jnp.stack(r, axis=0)
        ng = 8
        while ng > 2:
            h = ng // 2
            top = jnp.maximum(xs[:, :h], xs[:, h:])
            q = h // 2
            xs = jnp.concatenate(
                [_bitonic_merge(top[:, :q], True),
                 _bitonic_merge(top[:, q:], False)], axis=1)
            ng = h
        t = jnp.maximum(xs[:, 0], xs[:, 1])
        outs.append(jnp.sum(t, axis=0) * (1.0 / _K))
    return jnp.concatenate(outs)


def _fused_body(x_ref, w_ref, b_ref, o_ref, pooled_ref):
    j = pl.program_id(0)
    bi = pl.program_id(1)
    pooled = _topk_pool(x_ref[0])
    pooled_ref[pl.ds(bi, 1), :] = pooled[None, :]

    @pl.when(bi == pl.num_programs(1) - 1)
    def _():
        partial = jax.lax.dot_general(
            pooled_ref[...], w_ref[...], (((1,), (1,)), ((), ())),
            preferred_element_type=jnp.float32)

        @pl.when(j == 0)
        def _():
            o_ref[...] = partial + b_ref[...]

        @pl.when(j > 0)
        def _():
            o_ref[...] = o_ref[...] + partial


_H_SC = 512
_CPW = 16


def _sc_topk(x4):
    nb, seq = x4.shape[0], x4.shape[1]
    base_blk = (x4.shape[2] * _CPW - _H_SC) // _CPW
    mesh = plsc.VectorSubcoreMesh(core_axis_name="c", subcore_axis_name="s")

    @functools.partial(
        pl.kernel,
        out_type=jax.ShapeDtypeStruct((nb, _H_SC // _CPW, _CPW), jnp.float32),
        mesh=mesh,
        compiler_params=pltpu.CompilerParams(use_tc_tiling_on_sc=False),
        scratch_types=[
            pltpu.VMEM((seq, _CPW), jnp.float32),
            pltpu.VMEM((nb, _CPW), jnp.float32),
        ],
    )
    def sc_k(x_hbm, out_hbm, buf, outv):
        wid = lax.axis_index("s") * 2 + lax.axis_index("c")
        for b in range(nb):
            pltpu.sync_copy(x_hbm.at[b, :, base_blk + wid, :], buf)

            def group(g, desc):
                v = [buf[g * _K + p] for p in range(_K)]
                _sort32_list(v, desc)
                return v

            def body(g, carry):
                chunk = group(g, False)
                t = [jnp.maximum(carry[p], chunk[p]) for p in range(_K)]
                _cleanup_list(t, True)
                return tuple(t)

            fin = lax.fori_loop(1, seq // _K, body, tuple(group(0, True)))
            tot = fin[0]
            for p in range(1, _K):
                tot = tot + fin[p]
            outv[b] = tot * (1.0 / _K)
        pltpu.sync_copy(outv, out_hbm.at[:, wid, :])

    return sc_k(x4)


def _combine_body(a_ref, p_ref, w2_ref, o_ref):
    part = jax.lax.dot_general(
        p_ref[...], w2_ref[...], (((1,), (1,)), ((), ())),
        preferred_element_type=jnp.float32)
    o_ref[...] = jnp.tanh(a_ref[...] + part)


@jax.jit
def kernel(hidden_states, W, b):
    bsz, seq, hid = hidden_states.shape
    hb = 512
    b2 = b.reshape(1, hid)
    ntc = (hid - _H_SC) // hb

    pooled_sc = _sc_topk(
        hidden_states.reshape(bsz, seq, hid // _CPW, _CPW)
    ).reshape(bsz, _H_SC)

    acc = pl.pallas_call(
        _fused_body,
        grid=(ntc, bsz),
        in_specs=[
            pl.BlockSpec((1, seq, hb), lambda j, i: (i, 0, j)),
            pl.BlockSpec((hid, hb), lambda j, i: (0, j)),
            pl.BlockSpec((1, hid), lambda j, i: (0, 0)),
        ],
        out_specs=pl.BlockSpec((bsz, hid), lambda j, i: (0, 0)),
        out_shape=jax.ShapeDtypeStruct((bsz, hid), jnp.float32),
        scratch_shapes=[pltpu.VMEM((4, hb), jnp.float32)],
    )(hidden_states, W, b2)

    out = pl.pallas_call(
        _combine_body,
        grid=(1,),
        in_specs=[
            pl.BlockSpec((bsz, hid), lambda i: (0, 0)),
            pl.BlockSpec((bsz, _H_SC), lambda i: (0, 0)),
            pl.BlockSpec((hid, _H_SC), lambda i: (0, hid // _H_SC - 1)),
        ],
        out_specs=pl.BlockSpec((bsz, hid), lambda i: (0, 0)),
        out_shape=jax.ShapeDtypeStruct((bsz, hid), jnp.float32),
    )(acc, pooled_sc, W)
    return out


# speedup vs baseline: 4.8237x; 4.8237x over previous
import functools

import jax
import jax.numpy as jnp
from jax import lax
from jax.experimental import pallas as pl
from jax.experimental.pallas import tpu as pltpu
from jax.experimental.pallas import tpu_sc as plsc

_K = 32
_SEQ = 2048
_GROUPS = _SEQ // _K


def _ce(v, i, l, desc):
    a, b = v[i], v[l]
    mx = jnp.maximum(a, b)
    mn = jnp.minimum(a, b)
    if desc:
        v[i], v[l] = mx, mn
    else:
        v[i], v[l] = mn, mx


def _oem_merge(v, lo, n, r, desc):
    m = r * 2
    if m < n:
        _oem_merge(v, lo, n, m, desc)
        _oem_merge(v, lo + r, n, m, desc)
        for i in range(lo + r, lo + n - r, m):
            _ce(v, i, i + r, desc)
    else:
        _ce(v, lo, lo + r, desc)


def _sort32_list(v, desc, lo=0, n=_K):
    if n > 1:
        m = n // 2
        _sort32_list(v, desc, lo, m)
        _sort32_list(v, desc, lo + m, m)
        _oem_merge(v, lo, n, 1, desc)


def _cleanup_list(v, desc):
    j = _K // 2
    while j >= 1:
        for i in range(_K):
            l = i ^ j
            if l > i:
                a, b = v[i], v[l]
                mx = jnp.maximum(a, b)
                mn = jnp.minimum(a, b)
                if desc:
                    v[i], v[l] = mx, mn
                else:
                    v[i], v[l] = mn, mx
        j //= 2


def _cleanup_list_masked(v, dmask):
    j = _K // 2
    while j >= 1:
        for i in range(_K):
            l = i ^ j
            if l > i:
                a, b = v[i], v[l]
                mx = jnp.maximum(a, b)
                mn = jnp.minimum(a, b)
                v[i] = jnp.where(dmask, mx, mn)
                v[l] = jnp.where(dmask, mn, mx)
        j //= 2


def _merge_lists(va, vb, desc):
    t = [jnp.maximum(va[p], vb[p]) for p in range(_K)]
    _cleanup_list(t, desc)
    return t


def _ce_split(x, j):
    g = _K // (2 * j)
    xr = x.reshape((g, 2, j) + x.shape[1:])
    return xr[:, 0], xr[:, 1], g


def _bitonic_merge(x, desc):
    j = _K // 2
    while j >= 1:
        a, b, g = _ce_split(x, j)
        mn = jnp.minimum(a, b)
        mx = jnp.maximum(a, b)
        parts = []
        for gi in range(g):
            if desc:
                parts.append(mx[gi])
                parts.append(mn[gi])
            else:
                parts.append(mn[gi])
                parts.append(mx[gi])
        x = jnp.concatenate(parts, axis=0)
        j //= 2
    return x


def _topk_pool(x):
    hb = x.shape[-1]
    x = x.reshape(_K, _GROUPS, hb)
    dmask = jax.lax.broadcasted_iota(jnp.int32, (8, 128), 0) < 4
    pmask = jax.lax.broadcasted_iota(jnp.int32, (16, 128), 0) < 8
    outs = []
    for c in range(hb // 128):
        def ptile(p, gta, gtb):
            t = x[p, gta * 8:(gta + 1) * 8, c * 128:(c + 1) * 128]
            u = x[p, gtb * 8:(gtb + 1) * 8, c * 128:(c + 1) * 128]
            return jnp.concatenate([t, u], axis=0).astype(jnp.bfloat16)

        def sorted_pack(gta, gtb, desc):
            v = [ptile(p, gta, gtb) for p in range(_K)]
            _sort32_list(v, desc)
            return v

        q01 = _merge_lists(sorted_pack(0, 1, True), sorted_pack(4, 5, False),
                           True)
        q23 = _merge_lists(sorted_pack(2, 3, True), sorted_pack(6, 7, False),
                           False)
        r = [jnp.maximum(q01[p], q23[p]) for p in range(_K)]
        _cleanup_list_masked(r, pmask)
        r = [jnp.maximum(r[p][:8], r[p][8:]).astype(jnp.float32)
             for p in range(_K)]
        _cleanup_list_masked(r, dmask)
        xs = jnp.stack(r, axis=0)
        ng = 8
        while ng > 2:
            h = ng // 2
            top = jnp.maximum(xs[:, :h], xs[:, h:])
            q = h // 2
            xs = jnp.concatenate(
                [_bitonic_merge(top[:, :q], True),
                 _bitonic_merge(top[:, q:], False)], axis=1)
            ng = h
        t = jnp.maximum(xs[:, 0], xs[:, 1])
        outs.append(jnp.sum(t, axis=0) * (1.0 / _K))
    return jnp.concatenate(outs)


def _fused_body(x_ref, w_ref, b_ref, o_ref, pooled_ref):
    j = pl.program_id(0)
    bi = pl.program_id(1)
    pooled = _topk_pool(x_ref[0])
    pooled_ref[pl.ds(bi, 1), :] = pooled[None, :]

    @pl.when(bi == pl.num_programs(1) - 1)
    def _():
        partial = jax.lax.dot_general(
            pooled_ref[...], w_ref[...], (((1,), (1,)), ((), ())),
            preferred_element_type=jnp.float32)

        @pl.when(j == 0)
        def _():
            o_ref[...] = partial + b_ref[...]

        @pl.when(j > 0)
        def _():
            o_ref[...] = o_ref[...] + partial


_H_SC = 512
_CPW = 16


def _sc_topk(x4):
    nb, seq, hid = x4.shape
    base = hid - _H_SC
    mesh = plsc.VectorSubcoreMesh(core_axis_name="c", subcore_axis_name="s")

    @functools.partial(
        pl.kernel,
        out_type=jax.ShapeDtypeStruct((nb, _H_SC), jnp.float32),
        mesh=mesh,
        compiler_params=pltpu.CompilerParams(use_tc_tiling_on_sc=False),
        scratch_types=[
            pltpu.VMEM((seq, _CPW), jnp.float32),
            pltpu.VMEM((nb, _CPW), jnp.float32),
        ],
    )
    def sc_k(x_hbm, out_hbm, buf, outv):
        wid = lax.axis_index("s") * 2 + lax.axis_index("c")
        for b in range(nb):
            pltpu.sync_copy(x_hbm.at[b, :, pl.ds(base + wid * _CPW, _CPW)],
                            buf)

            def group(g, desc):
                v = [buf[g * _K + p] for p in range(_K)]
                _sort32_list(v, desc)
                return v

            def body(g, carry):
                chunk = group(g, False)
                t = [jnp.maximum(carry[p], chunk[p]) for p in range(_K)]
                _cleanup_list(t, True)
                return tuple(t)

            fin = lax.fori_loop(1, seq // _K, body, tuple(group(0, True)))
            tot = fin[0]
            for p in range(1, _K):
                tot = tot + fin[p]
            outv[b] = tot * (1.0 / _K)
        pltpu.sync_copy(outv, out_hbm.at[:, pl.ds(wid * _CPW, _CPW)])

    return sc_k(x4)


def _combine_body(a_ref, p_ref, w2_ref, o_ref):
    part = jax.lax.dot_general(
        p_ref[...], w2_ref[...], (((1,), (1,)), ((), ())),
        preferred_element_type=jnp.float32)
    o_ref[...] = jnp.tanh(a_ref[...] + part)


@jax.jit
def kernel(hidden_states, W, b):
    bsz, seq, hid = hidden_states.shape
    hb = 512
    b2 = b.reshape(1, hid)
    ntc = (hid - _H_SC) // hb

    pooled_sc = _sc_topk(hidden_states)

    acc = pl.pallas_call(
        _fused_body,
        grid=(ntc, bsz),
        in_specs=[
            pl.BlockSpec((1, seq, hb), lambda j, i: (i, 0, j)),
            pl.BlockSpec((hid, hb), lambda j, i: (0, j)),
            pl.BlockSpec((1, hid), lambda j, i: (0, 0)),
        ],
        out_specs=pl.BlockSpec((bsz, hid), lambda j, i: (0, 0)),
        out_shape=jax.ShapeDtypeStruct((bsz, hid), jnp.float32),
        scratch_shapes=[pltpu.VMEM((4, hb), jnp.float32)],
    )(hidden_states, W, b2)

    out = pl.pallas_call(
        _combine_body,
        grid=(1,),
        in_specs=[
            pl.BlockSpec((bsz, hid), lambda i: (0, 0)),
            pl.BlockSpec((bsz, _H_SC), lambda i: (0, 0)),
            pl.BlockSpec((hid, _H_SC), lambda i: (0, hid // _H_SC - 1)),
        ],
        out_specs=pl.BlockSpec((bsz, hid), lambda i: (0, 0)),
        out_shape=jax.ShapeDtypeStruct((bsz, hid), jnp.float32),
    )(acc, pooled_sc, W)
    return out


# speedup vs baseline: 10.5141x; 2.1797x over previous
import jax
import jax.numpy as jnp
from jax.experimental import pallas as pl
from jax.experimental.pallas import tpu as pltpu

_K = 32
_SEQ = 2048
_GROUPS = _SEQ // _K


def _ce(v, i, l, desc):
    a, b = v[i], v[l]
    mx = jnp.maximum(a, b)
    mn = jnp.minimum(a, b)
    if desc:
        v[i], v[l] = mx, mn
    else:
        v[i], v[l] = mn, mx


def _oem_merge(v, lo, n, r, desc):
    m = r * 2
    if m < n:
        _oem_merge(v, lo, n, m, desc)
        _oem_merge(v, lo + r, n, m, desc)
        for i in range(lo + r, lo + n - r, m):
            _ce(v, i, i + r, desc)
    else:
        _ce(v, lo, lo + r, desc)


def _sort32_list(v, desc, lo=0, n=_K):
    if n > 1:
        m = n // 2
        _sort32_list(v, desc, lo, m)
        _sort32_list(v, desc, lo + m, m)
        _oem_merge(v, lo, n, 1, desc)


def _cleanup_list(v, desc):
    j = _K // 2
    while j >= 1:
        for i in range(_K):
            l = i ^ j
            if l > i:
                a, b = v[i], v[l]
                mx = jnp.maximum(a, b)
                mn = jnp.minimum(a, b)
                if desc:
                    v[i], v[l] = mx, mn
                else:
                    v[i], v[l] = mn, mx
        j //= 2


def _cleanup_list_masked(v, dmask):
    j = _K // 2
    while j >= 1:
        for i in range(_K):
            l = i ^ j
            if l > i:
                a, b = v[i], v[l]
                mx = jnp.maximum(a, b)
                mn = jnp.minimum(a, b)
                v[i] = jnp.where(dmask, mx, mn)
                v[l] = jnp.where(dmask, mn, mx)
        j //= 2


def _merge_lists(va, vb, desc):
    t = [jnp.maximum(va[p], vb[p]) for p in range(_K)]
    _cleanup_list(t, desc)
    return t


def _ce_split(x, j):
    g = _K // (2 * j)
    xr = x.reshape((g, 2, j) + x.shape[1:])
    return xr[:, 0], xr[:, 1], g


def _bitonic_merge(x, desc):
    j = _K // 2
    while j >= 1:
        a, b, g = _ce_split(x, j)
        mn = jnp.minimum(a, b)
        mx = jnp.maximum(a, b)
        parts = []
        for gi in range(g):
            if desc:
                parts.append(mx[gi])
                parts.append(mn[gi])
            else:
                parts.append(mn[gi])
                parts.append(mx[gi])
        x = jnp.concatenate(parts, axis=0)
        j //= 2
    return x


def _topk_pool(x):
    hb = x.shape[-1]
    x = x.reshape(_K, _GROUPS, hb)
    dmask = jax.lax.broadcasted_iota(jnp.int32, (8, 128), 0) < 4
    pmask = jax.lax.broadcasted_iota(jnp.int32, (16, 128), 0) < 8
    outs = []
    for c in range(hb // 128):
        def ptile(p, gta, gtb):
            t = x[p, gta * 8:(gta + 1) * 8, c * 128:(c + 1) * 128]
            u = x[p, gtb * 8:(gtb + 1) * 8, c * 128:(c + 1) * 128]
            return jnp.concatenate([t, u], axis=0).astype(jnp.bfloat16)

        def sorted_pack(gta, gtb, desc):
            v = [ptile(p, gta, gtb) for p in range(_K)]
            _sort32_list(v, desc)
            return v

        q01 = _merge_lists(sorted_pack(0, 1, True), sorted_pack(4, 5, False),
                           True)
        q23 = _merge_lists(sorted_pack(2, 3, True), sorted_pack(6, 7, False),
                           False)
        r = [jnp.maximum(q01[p], q23[p]) for p in range(_K)]
        _cleanup_list_masked(r, pmask)
        r = [jnp.maximum(r[p][:8], r[p][8:]).astype(jnp.float32)
             for p in range(_K)]
        _cleanup_list_masked(r, dmask)
        xs = jnp.stack(r, axis=0)
        ng = 8
        while ng > 2:
            h = ng // 2
            top = jnp.maximum(xs[:, :h], xs[:, h:])
            q = h // 2
            xs = jnp.concatenate(
                [_bitonic_merge(top[:, :q], True),
                 _bitonic_merge(top[:, q:], False)], axis=1)
            ng = h
        t = jnp.maximum(xs[:, 0], xs[:, 1])
        outs.append(jnp.sum(t, axis=0) * (1.0 / _K))
    return jnp.concatenate(outs)


def _fused_body(x_ref, w_ref, b_ref, o_ref, pooled_ref):
    j = pl.program_id(0)
    bi = pl.program_id(1)
    nj = pl.num_programs(0)
    pooled = _topk_pool(x_ref[0])
    pooled_ref[pl.ds(bi, 1), :] = pooled[None, :]

    @pl.when(bi == pl.num_programs(1) - 1)
    def _():
        partial = jax.lax.dot_general(
            pooled_ref[...], w_ref[...], (((1,), (1,)), ((), ())),
            preferred_element_type=jnp.float32)

        @pl.when(j == 0)
        def _():
            o_ref[...] = partial + b_ref[...]

        @pl.when(jnp.logical_and(j > 0, j < nj - 1))
        def _():
            o_ref[...] = o_ref[...] + partial

        @pl.when(j == nj - 1)
        def _():
            o_ref[...] = jnp.tanh(o_ref[...] + partial)


@jax.jit
def kernel(hidden_states, W, b):
    bsz, seq, hid = hidden_states.shape
    hb = 512
    b2 = b.reshape(1, hid)
    out = pl.pallas_call(
        _fused_body,
        grid=(hid // hb, bsz),
        in_specs=[
            pl.BlockSpec((1, seq, hb), lambda j, i: (i, 0, j)),
            pl.BlockSpec((hid, hb), lambda j, i: (0, j)),
            pl.BlockSpec((1, hid), lambda j, i: (0, 0)),
        ],
        out_specs=pl.BlockSpec((bsz, hid), lambda j, i: (0, 0)),
        out_shape=jax.ShapeDtypeStruct((bsz, hid), jnp.float32),
        scratch_shapes=[pltpu.VMEM((4, hb), jnp.float32)],
    )(hidden_states, W, b2)
    return out


# speedup vs baseline: 11.0524x; 1.0512x over previous
import jax
import jax.numpy as jnp
from jax.experimental import pallas as pl
from jax.experimental.pallas import tpu as pltpu

_K = 32
_SEQ = 2048
_GROUPS = _SEQ // _K


def _ce(v, i, l, desc):
    a, b = v[i], v[l]
    mx = jnp.maximum(a, b)
    mn = jnp.minimum(a, b)
    if desc:
        v[i], v[l] = mx, mn
    else:
        v[i], v[l] = mn, mx


def _oem_merge(v, lo, n, r, desc):
    m = r * 2
    if m < n:
        _oem_merge(v, lo, n, m, desc)
        _oem_merge(v, lo + r, n, m, desc)
        for i in range(lo + r, lo + n - r, m):
            _ce(v, i, i + r, desc)
    else:
        _ce(v, lo, lo + r, desc)


def _sort32_list(v, desc, lo=0, n=_K):
    if n > 1:
        m = n // 2
        _sort32_list(v, desc, lo, m)
        _sort32_list(v, desc, lo + m, m)
        _oem_merge(v, lo, n, 1, desc)


def _cleanup_list(v, desc):
    j = _K // 2
    while j >= 1:
        for i in range(_K):
            l = i ^ j
            if l > i:
                a, b = v[i], v[l]
                mx = jnp.maximum(a, b)
                mn = jnp.minimum(a, b)
                if desc:
                    v[i], v[l] = mx, mn
                else:
                    v[i], v[l] = mn, mx
        j //= 2


def _merge_lists(va, vb):
    t = [jnp.maximum(va[p], vb[_K - 1 - p]) for p in range(_K)]
    _cleanup_list(t, True)
    return t


def _topk_pool(x):
    hb = x.shape[-1]
    x = x.reshape(_K, _GROUPS, hb)
    outs = []
    for c in range(hb // 128):
        def ptile(p, gta, gtb):
            t = x[p, gta * 8:(gta + 1) * 8, c * 128:(c + 1) * 128]
            u = x[p, gtb * 8:(gtb + 1) * 8, c * 128:(c + 1) * 128]
            return jnp.concatenate([t, u], axis=0).astype(jnp.bfloat16)

        def sorted_pack(gta, gtb):
            v = [ptile(p, gta, gtb) for p in range(_K)]
            _sort32_list(v, True)
            return v

        q01 = _merge_lists(sorted_pack(0, 1), sorted_pack(4, 5))
        q23 = _merge_lists(sorted_pack(2, 3), sorted_pack(6, 7))
        r = _merge_lists(q01, q23)
        t = [jnp.maximum(r[p][:8], r[_K - 1 - p][8:]).astype(jnp.float32)
             for p in range(_K)]
        _cleanup_list(t, True)
        u = [jnp.maximum(t[p][:4], t[_K - 1 - p][4:]) for p in range(_K)]
        _cleanup_list(u, True)
        w = [jnp.maximum(u[p][:2], u[_K - 1 - p][2:]) for p in range(_K)]
        _cleanup_list(w, True)
        z = [jnp.maximum(w[p][:1], w[_K - 1 - p][1:]) for p in range(_K)]
        tot = z[0]
        for p in range(1, _K):
            tot = tot + z[p]
        outs.append(tot[0] * (1.0 / _K))
    return jnp.concatenate(outs)


def _fused_body(x_ref, w_ref, b_ref, o_ref, pooled_ref):
    j = pl.program_id(0)
    bi = pl.program_id(1)
    nj = pl.num_programs(0)
    pooled = _topk_pool(x_ref[0])
    pooled_ref[pl.ds(bi, 1), :] = pooled[None, :]

    @pl.when(bi == pl.num_programs(1) - 1)
    def _():
        partial = jax.lax.dot_general(
            pooled_ref[...], w_ref[...], (((1,), (1,)), ((), ())),
            preferred_element_type=jnp.float32)

        @pl.when(j == 0)
        def _():
            o_ref[...] = partial + b_ref[...]

        @pl.when(jnp.logical_and(j > 0, j < nj - 1))
        def _():
            o_ref[...] = o_ref[...] + partial

        @pl.when(j == nj - 1)
        def _():
            o_ref[...] = jnp.tanh(o_ref[...] + partial)


@jax.jit
def kernel(hidden_states, W, b):
    bsz, seq, hid = hidden_states.shape
    hb = 512
    b2 = b.reshape(1, hid)
    out = pl.pallas_call(
        _fused_body,
        grid=(hid // hb, bsz),
        in_specs=[
            pl.BlockSpec((1, seq, hb), lambda j, i: (i, 0, j)),
            pl.BlockSpec((hid, hb), lambda j, i: (0, j)),
            pl.BlockSpec((1, hid), lambda j, i: (0, 0)),
        ],
        out_specs=pl.BlockSpec((bsz, hid), lambda j, i: (0, 0)),
        out_shape=jax.ShapeDtypeStruct((bsz, hid), jnp.float32),
        scratch_shapes=[pltpu.VMEM((4, hb), jnp.float32)],
    )(hidden_states, W, b2)
    return out
